# baseline (speedup 1.0000x reference)
import jax
import jax.numpy as jnp
from jax.experimental import pallas as pl
from jax.experimental.pallas import tpu as pltpu


def _copy_body(src_ref, out_ref, sem):
    copy = pltpu.make_async_copy(src_ref, out_ref, sem)
    copy.start()
    copy.wait()


def kernel(x, pos_emb):
    S = x.shape[0]
    D = pos_emb.shape[1]
    src = pos_emb[:S]
    out = pl.pallas_call(
        _copy_body,
        in_specs=[pl.BlockSpec(memory_space=pltpu.MemorySpace.HBM)],
        out_specs=pl.BlockSpec(memory_space=pltpu.MemorySpace.HBM),
        out_shape=jax.ShapeDtypeStruct((S, D), jnp.float32),
        scratch_shapes=[pltpu.SemaphoreType.DMA],
    )(src)
    return out.reshape(S, 1, D)
